# baseline (speedup 1.0000x reference)
import functools

import numpy as np
import jax
import jax.numpy as jnp
from jax import lax
from jax.experimental import pallas as pl
from jax.experimental.pallas import tpu as pltpu
from jax.experimental.pallas import tpu_sc as plsc

N = 10000
D = 128
E = 320000
NC = 2
NS = 16
NW = NC * NS
NPAD = 10240
RPT = NPAD // NS
K = 125
NCHUNK = (E // NW) // K
G = 16
DEG_W = 128

ALPHA = 0.5
BETA1 = float(np.log(2.0))
BETA2 = float(np.log(1.5))


@functools.cache
def _sc_kernels():
    mesh = plsc.VectorSubcoreMesh(
        core_axis_name="c", subcore_axis_name="s", num_cores=NC, num_subcores=NS
    )

    @functools.partial(
        pl.kernel,
        out_type=jax.ShapeDtypeStruct((NC, NPAD, DEG_W), jnp.float32),
        mesh=mesh,
        scratch_types=[
            pltpu.VMEM((G, K), jnp.int32),
            pltpu.VMEM((K, DEG_W), jnp.float32),
            pltpu.VMEM_SHARED((NPAD, DEG_W), jnp.float32),
            pltpu.SemaphoreType.DMA,
        ],
    )
    def deg_call(dst3, ones_hbm, zeros_hbm, out, dst_v, ones_v, acc, ssem):
        cid = lax.axis_index("c")
        sid = lax.axis_index("s")
        wid = sid * NC + cid
        pltpu.sync_copy(zeros_hbm, acc.at[pl.ds(sid * RPT, RPT)])
        pltpu.sync_copy(ones_hbm, ones_v)
        plsc.subcore_barrier()

        def group(jg, c):
            pltpu.sync_copy(dst3.at[wid, pl.ds(jg * G, G)], dst_v)

            def fire(r, c2):
                pltpu.async_copy(ones_v, acc.at[dst_v.at[r]], ssem, add=True)
                return c2

            lax.fori_loop(0, G, fire, c)

            def drain(r, c2):
                pltpu.make_async_copy(ones_v, acc.at[dst_v.at[r]], ssem).wait()
                return c2

            return lax.fori_loop(0, G, drain, c)

        lax.fori_loop(0, NCHUNK // G, group, 0)
        plsc.subcore_barrier()
        pltpu.sync_copy(acc.at[pl.ds(sid * RPT, RPT)],
                        out.at[cid, pl.ds(sid * RPT, RPT)])

    @functools.partial(
        pl.kernel,
        out_type=jax.ShapeDtypeStruct((NC, NPAD, D), jnp.float32),
        mesh=mesh,
        scratch_types=[
            pltpu.VMEM((G, K), jnp.int32),
            pltpu.VMEM((G, K), jnp.int32),
            pltpu.VMEM((K, D), jnp.float32),
            pltpu.VMEM((K, D), jnp.float32),
            pltpu.VMEM_SHARED((NPAD, D), jnp.float32),
            pltpu.SemaphoreType.DMA,
            pltpu.SemaphoreType.DMA,
            pltpu.SemaphoreType.DMA,
            pltpu.SemaphoreType.DMA,
        ],
    )
    def prop_call(src3, dst3, g_hbm, zeros_hbm, out,
                  src_v, dst_v, buf0, buf1, acc, sem0, sem1, ssem0, ssem1):
        cid = lax.axis_index("c")
        sid = lax.axis_index("s")
        wid = sid * NC + cid
        pltpu.sync_copy(zeros_hbm, acc.at[pl.ds(sid * RPT, RPT)])
        plsc.subcore_barrier()

        def group(jg, c):
            pltpu.sync_copy(src3.at[wid, pl.ds(jg * G, G)], src_v)
            pltpu.sync_copy(dst3.at[wid, pl.ds(jg * G, G)], dst_v)
            pltpu.async_copy(g_hbm.at[src_v.at[0]], buf0, sem0)
            pltpu.async_copy(g_hbm.at[src_v.at[1]], buf1, sem1)

            def step(r2, c2):
                r = 2 * r2
                pltpu.make_async_copy(g_hbm.at[src_v.at[r]], buf0, sem0).wait()
                pltpu.async_copy(buf0, acc.at[dst_v.at[r]], ssem0, add=True)

                pltpu.make_async_copy(g_hbm.at[src_v.at[r + 1]], buf1, sem1).wait()
                pltpu.async_copy(buf1, acc.at[dst_v.at[r + 1]], ssem1, add=True)

                pltpu.make_async_copy(buf0, acc.at[dst_v.at[r]], ssem0).wait()

                @pl.when(r + 2 < G)
                def _():
                    pltpu.async_copy(g_hbm.at[src_v.at[r + 2]], buf0, sem0)

                pltpu.make_async_copy(buf1, acc.at[dst_v.at[r + 1]], ssem1).wait()

                @pl.when(r + 3 < G)
                def _():
                    pltpu.async_copy(g_hbm.at[src_v.at[r + 3]], buf1, sem1)

                return c2

            return lax.fori_loop(0, G // 2, step, c)

        lax.fori_loop(0, NCHUNK // G, group, 0)
        plsc.subcore_barrier()
        pltpu.sync_copy(acc.at[pl.ds(sid * RPT, RPT)],
                        out.at[cid, pl.ds(sid * RPT, RPT)])

    return deg_call, prop_call


R = 1000


def _dinv_block(deg_ref):
    deg = deg_ref[0, :, :1] + deg_ref[1, :, :1] + 1.0
    return lax.rsqrt(deg)


def _prologue_body(x_ref, w1_ref, b1_ref, deg_ref, x0_ref, g1_ref):
    x0 = jnp.dot(x_ref[...], w1_ref[...],
                 preferred_element_type=jnp.float32) + b1_ref[...]
    x0_ref[...] = x0
    g1_ref[...] = x0 * _dinv_block(deg_ref)


def _layer_body(beta, final, agg_ref, gin_ref, x0_ref, deg_ref, wc_ref, *rest):
    dinv = _dinv_block(deg_ref)
    agg = (agg_ref[0] + agg_ref[1] + gin_ref[...]) * dinv
    z = (1.0 - ALPHA) * agg + ALPHA * x0_ref[...]
    h = (1.0 - beta) * z + beta * jnp.dot(
        z, wc_ref[...], preferred_element_type=jnp.float32)
    h = jnp.maximum(h, 0.0)
    if final:
        w2_ref, b2_ref, out_ref = rest
        out_ref[...] = jnp.dot(
            h, w2_ref[...], preferred_element_type=jnp.float32) + b2_ref[...]
    else:
        (gout_ref,) = rest
        gout_ref[...] = h * dinv


_row_spec = pl.BlockSpec((R, D), lambda i: (i, 0))
_mat_spec = pl.BlockSpec((D, D), lambda i: (0, 0))
_bias_spec = pl.BlockSpec((1, D), lambda i: (0, 0))
_deg_spec = pl.BlockSpec((2, R, DEG_W), lambda i: (0, i, 0))
_agg_spec = pl.BlockSpec((2, R, D), lambda i: (0, i, 0))

_prologue = pl.pallas_call(
    _prologue_body,
    grid=(N // R,),
    in_specs=[_row_spec, _mat_spec, _bias_spec, _deg_spec],
    out_specs=[_row_spec, _row_spec],
    out_shape=[jax.ShapeDtypeStruct((N, D), jnp.float32)] * 2,
)

_layer1 = pl.pallas_call(
    functools.partial(_layer_body, BETA1, False),
    grid=(N // R,),
    in_specs=[_agg_spec, _row_spec, _row_spec, _deg_spec, _mat_spec],
    out_specs=_row_spec,
    out_shape=jax.ShapeDtypeStruct((N, D), jnp.float32),
)

_layer2 = pl.pallas_call(
    functools.partial(_layer_body, BETA2, True),
    grid=(N // R,),
    in_specs=[_agg_spec, _row_spec, _row_spec, _deg_spec, _mat_spec,
              _mat_spec, _bias_spec],
    out_specs=_row_spec,
    out_shape=jax.ShapeDtypeStruct((N, D), jnp.float32),
)


def kernel(x, edge_index, W1, b1, Wc1, Wc2, W2, b2):
    src3 = edge_index[0].reshape(NW, NCHUNK, K)
    dst3 = edge_index[1].reshape(NW, NCHUNK, K)
    ones_deg = jnp.ones((K, DEG_W), jnp.float32)
    zeros_deg = jnp.zeros((RPT, DEG_W), jnp.float32)
    zeros_rows = jnp.zeros((RPT, D), jnp.float32)

    deg_call, prop_call = _sc_kernels()
    deg2 = deg_call(dst3, ones_deg, zeros_deg)
    x0, g1 = _prologue(x, W1, b1.reshape(1, D), deg2)
    agg1 = prop_call(src3, dst3, g1, zeros_rows)
    g2 = _layer1(agg1, g1, x0, deg2, Wc1)
    agg2 = prop_call(src3, dst3, g2, zeros_rows)
    logits = _layer2(agg2, g2, x0, deg2, Wc2, W2, b2.reshape(1, D))
    return logits


# speedup vs baseline: 1.2210x; 1.2210x over previous
import functools

import numpy as np
import jax
import jax.numpy as jnp
from jax import lax
from jax.experimental import pallas as pl
from jax.experimental.pallas import tpu as pltpu
from jax.experimental.pallas import tpu_sc as plsc

N = 10000
D = 128
E = 320000
NC = 2
NS = 16
NW = NC * NS
NPAD = 10240
RPT = NPAD // NS
K = 125
NCHUNK = (E // NW) // K
G = 40
DEG_W = 128

ALPHA = 0.5
BETA1 = float(np.log(2.0))
BETA2 = float(np.log(1.5))


@functools.cache
def _sc_kernels():
    mesh = plsc.VectorSubcoreMesh(
        core_axis_name="c", subcore_axis_name="s", num_cores=NC, num_subcores=NS
    )

    @functools.partial(
        pl.kernel,
        out_type=jax.ShapeDtypeStruct((NC, NPAD, DEG_W), jnp.float32),
        mesh=mesh,
        scratch_types=[
            pltpu.VMEM((G, K), jnp.int32),
            pltpu.VMEM((K, DEG_W), jnp.float32),
            pltpu.VMEM_SHARED((NPAD, DEG_W), jnp.float32),
            pltpu.SemaphoreType.DMA,
        ],
    )
    def deg_call(dst3, ones_hbm, zeros_hbm, out, dst_v, ones_v, acc, ssem):
        cid = lax.axis_index("c")
        sid = lax.axis_index("s")
        wid = sid * NC + cid
        pltpu.sync_copy(zeros_hbm, acc.at[pl.ds(sid * RPT, RPT)])
        pltpu.sync_copy(ones_hbm, ones_v)
        plsc.subcore_barrier()

        def group(jg, c):
            pltpu.sync_copy(dst3.at[wid, pl.ds(jg * G, G)], dst_v)

            def fire(r, c2):
                pltpu.async_copy(ones_v, acc.at[dst_v.at[r]], ssem, add=True)
                return c2

            lax.fori_loop(0, G, fire, c)

            def drain(r, c2):
                pltpu.make_async_copy(ones_v, acc.at[dst_v.at[r]], ssem).wait()
                return c2

            return lax.fori_loop(0, G, drain, c)

        lax.fori_loop(0, NCHUNK // G, group, 0)
        plsc.subcore_barrier()
        pltpu.sync_copy(acc.at[pl.ds(sid * RPT, RPT)],
                        out.at[cid, pl.ds(sid * RPT, RPT)])

    @functools.partial(
        pl.kernel,
        out_type=jax.ShapeDtypeStruct((NC, NPAD, D), jnp.float32),
        mesh=mesh,
        scratch_types=[
            pltpu.VMEM((G, K), jnp.int32),
            pltpu.VMEM((G, K), jnp.int32),
            pltpu.VMEM((K, D), jnp.float32),
            pltpu.VMEM((K, D), jnp.float32),
            pltpu.VMEM_SHARED((NPAD, D), jnp.float32),
            pltpu.SemaphoreType.DMA,
            pltpu.SemaphoreType.DMA,
        ],
    )
    def prop_call(src3, dst3, g_hbm, zeros_hbm, out,
                  src_v, dst_v, buf0, buf1, acc, sem0, sem1):
        cid = lax.axis_index("c")
        sid = lax.axis_index("s")
        wid = sid * NC + cid
        pltpu.sync_copy(zeros_hbm, acc.at[pl.ds(sid * RPT, RPT)])
        plsc.subcore_barrier()

        def group(jg, c):
            pltpu.sync_copy(src3.at[wid, pl.ds(jg * G, G)], src_v)
            pltpu.sync_copy(dst3.at[wid, pl.ds(jg * G, G)], dst_v)
            pltpu.async_copy(g_hbm.at[src_v.at[0]], buf0, sem0)
            pltpu.async_copy(g_hbm.at[src_v.at[1]], buf1, sem1)

            def step(r2, c2):
                r = 2 * r2
                pltpu.make_async_copy(g_hbm.at[src_v.at[r]], buf0, sem0).wait()
                pltpu.sync_copy(buf0, acc.at[dst_v.at[r]], add=True)

                @pl.when(r + 2 < G)
                def _():
                    pltpu.async_copy(g_hbm.at[src_v.at[r + 2]], buf0, sem0)

                pltpu.make_async_copy(g_hbm.at[src_v.at[r + 1]], buf1, sem1).wait()
                pltpu.sync_copy(buf1, acc.at[dst_v.at[r + 1]], add=True)

                @pl.when(r + 3 < G)
                def _():
                    pltpu.async_copy(g_hbm.at[src_v.at[r + 3]], buf1, sem1)

                return c2

            return lax.fori_loop(0, G // 2, step, c)

        lax.fori_loop(0, NCHUNK // G, group, 0)
        plsc.subcore_barrier()
        pltpu.sync_copy(acc.at[pl.ds(sid * RPT, RPT)],
                        out.at[cid, pl.ds(sid * RPT, RPT)])

    return deg_call, prop_call


R = 1000


def _dinv_block(deg_ref):
    deg = deg_ref[0, :, :1] + deg_ref[1, :, :1] + 1.0
    return lax.rsqrt(deg)


def _linear1_body(x_ref, w1_ref, b1_ref, x0_ref):
    x0_ref[...] = jnp.dot(x_ref[...], w1_ref[...],
                          preferred_element_type=jnp.float32) + b1_ref[...]


def _scale_body(x0_ref, deg_ref, g1_ref):
    g1_ref[...] = x0_ref[...] * _dinv_block(deg_ref)


def _layer_body(beta, final, agg_ref, gin_ref, x0_ref, deg_ref, wc_ref, *rest):
    dinv = _dinv_block(deg_ref)
    agg = (agg_ref[0] + agg_ref[1] + gin_ref[...]) * dinv
    z = (1.0 - ALPHA) * agg + ALPHA * x0_ref[...]
    h = (1.0 - beta) * z + beta * jnp.dot(
        z, wc_ref[...], preferred_element_type=jnp.float32)
    h = jnp.maximum(h, 0.0)
    if final:
        w2_ref, b2_ref, out_ref = rest
        out_ref[...] = jnp.dot(
            h, w2_ref[...], preferred_element_type=jnp.float32) + b2_ref[...]
    else:
        (gout_ref,) = rest
        gout_ref[...] = h * dinv


_row_spec = pl.BlockSpec((R, D), lambda i: (i, 0))
_mat_spec = pl.BlockSpec((D, D), lambda i: (0, 0))
_bias_spec = pl.BlockSpec((1, D), lambda i: (0, 0))
_deg_spec = pl.BlockSpec((2, R, DEG_W), lambda i: (0, i, 0))
_agg_spec = pl.BlockSpec((2, R, D), lambda i: (0, i, 0))

_linear1 = pl.pallas_call(
    _linear1_body,
    grid=(N // R,),
    in_specs=[_row_spec, _mat_spec, _bias_spec],
    out_specs=_row_spec,
    out_shape=jax.ShapeDtypeStruct((N, D), jnp.float32),
)

_scale1 = pl.pallas_call(
    _scale_body,
    grid=(N // R,),
    in_specs=[_row_spec, _deg_spec],
    out_specs=_row_spec,
    out_shape=jax.ShapeDtypeStruct((N, D), jnp.float32),
)

_layer1 = pl.pallas_call(
    functools.partial(_layer_body, BETA1, False),
    grid=(N // R,),
    in_specs=[_agg_spec, _row_spec, _row_spec, _deg_spec, _mat_spec],
    out_specs=_row_spec,
    out_shape=jax.ShapeDtypeStruct((N, D), jnp.float32),
)

_layer2 = pl.pallas_call(
    functools.partial(_layer_body, BETA2, True),
    grid=(N // R,),
    in_specs=[_agg_spec, _row_spec, _row_spec, _deg_spec, _mat_spec,
              _mat_spec, _bias_spec],
    out_specs=_row_spec,
    out_shape=jax.ShapeDtypeStruct((N, D), jnp.float32),
)


def kernel(x, edge_index, W1, b1, Wc1, Wc2, W2, b2):
    src3 = edge_index[0].reshape(NW, NCHUNK, K)
    dst3 = edge_index[1].reshape(NW, NCHUNK, K)
    ones_deg = jnp.ones((K, DEG_W), jnp.float32)
    zeros_deg = jnp.zeros((RPT, DEG_W), jnp.float32)
    zeros_rows = jnp.zeros((RPT, D), jnp.float32)

    deg_call, prop_call = _sc_kernels()
    deg2 = deg_call(dst3, ones_deg, zeros_deg)
    x0 = _linear1(x, W1, b1.reshape(1, D))
    g1 = _scale1(x0, deg2)
    agg1 = prop_call(src3, dst3, g1, zeros_rows)
    g2 = _layer1(agg1, g1, x0, deg2, Wc1)
    agg2 = prop_call(src3, dst3, g2, zeros_rows)
    logits = _layer2(agg2, g2, x0, deg2, Wc2, W2, b2.reshape(1, D))
    return logits
